# baseline (speedup 1.0000x reference)
import functools

import jax
import jax.numpy as jnp
from jax import lax
from jax.experimental import pallas as pl
from jax.experimental.pallas import tpu as pltpu
from jax.experimental.pallas import tpu_sc as plsc

D_MODEL = 64
SCALE = 8.0
LANES = 16
NUM_CORES = 2
NUM_SUBCORES = 16
NUM_WORKERS = NUM_CORES * NUM_SUBCORES
CHUNK = 128
NBUF = 4
SUBLANES = 8


@functools.lru_cache(maxsize=None)
def _make_kernel(batch: int, seq: int):
    n_rows = batch * seq
    assert batch % CHUNK == 0
    i_blocks = batch // CHUNK
    n_blocks = n_rows // CHUNK
    assert n_blocks % (NUM_WORKERS * NBUF) == 0
    n_chunks = n_blocks // NUM_WORKERS
    n_rounds = n_chunks // NBUF
    d_blocks = D_MODEL // SUBLANES
    mesh = plsc.VectorSubcoreMesh(core_axis_name="c", subcore_axis_name="s")

    @functools.partial(
        pl.kernel,
        out_type=jax.ShapeDtypeStruct(
            (seq, d_blocks, i_blocks, SUBLANES, CHUNK), jnp.float32
        ),
        mesh=mesh,
        scratch_types=[
            pltpu.VMEM((n_chunks, CHUNK), jnp.int32),
            pltpu.VMEM((NBUF, CHUNK, D_MODEL), jnp.float32),
            pltpu.VMEM((NBUF, D_MODEL, CHUNK), jnp.float32),
        ]
        + [pltpu.SemaphoreType.DMA] * (2 * NBUF),
        compiler_params=pltpu.CompilerParams(
            use_tc_tiling_on_sc=False, needs_layout_passes=False
        ),
    )
    def emb_kernel(x_hbm, table_hbm, out_hbm, idx_v, rows_g, rows_s, *sems):
        gsem, ssem = sems[:NBUF], sems[NBUF:]
        wid = lax.axis_index("s") * NUM_CORES + lax.axis_index("c")
        base_block = wid * n_chunks
        pltpu.sync_copy(x_hbm.at[pl.ds(base_block, n_chunks)], idx_v)

        iota = lax.iota(jnp.int32, LANES)
        d_idx = [iota + jj * LANES for jj in range(D_MODEL // LANES)]

        def fire_gather(c, k):
            pltpu.async_copy(table_hbm.at[idx_v.at[c]], rows_g.at[k], gsem[k])

        def wait_gather(c, k):
            pltpu.make_async_copy(
                table_hbm.at[idx_v.at[c]], rows_g.at[k], gsem[k]
            ).wait()

        def out_slices(c, k):
            b = base_block + c
            j = b // i_blocks
            ihi = b % i_blocks
            return [
                (rows_s.at[k, pl.ds(dh * SUBLANES, SUBLANES)],
                 out_hbm.at[j, dh, ihi])
                for dh in range(d_blocks)
            ]

        def fire_scatter(c, k):
            for src, dst in out_slices(c, k):
                pltpu.async_copy(src, dst, ssem[k])

        def wait_scatter(c, k):
            for src, dst in out_slices(c, k):
                pltpu.make_async_copy(src, dst, ssem[k]).wait()

        def scale(k):
            slot = jnp.full((LANES,), k, jnp.int32)

            def row_body(i, carry):
                col = jnp.full((LANES,), i, jnp.int32)
                for jj in range(D_MODEL // LANES):
                    v = rows_g[k, i, pl.ds(jj * LANES, LANES)] * SCALE
                    plsc.store_scatter(rows_s, [slot, d_idx[jj], col], v)
                return carry

            lax.fori_loop(0, CHUNK, row_body, 0, unroll=4)

        def round_body(r, first, last):
            for k in range(NBUF):
                c = r * NBUF + k
                wait_gather(c, k)
                if not first:
                    wait_scatter(c - NBUF, k)
                scale(k)
                fire_scatter(c, k)
                if not last:
                    fire_gather(c + NBUF, k)

        for k in range(NBUF):
            fire_gather(k, k)
        round_body(0, first=True, last=False)
        lax.fori_loop(
            1,
            n_rounds - 1,
            lambda r, carry: (round_body(r, first=False, last=False), carry)[1],
            0,
        )
        round_body(n_rounds - 1, first=False, last=True)
        for k in range(NBUF):
            wait_scatter((n_rounds - 1) * NBUF + k, k)

    return emb_kernel


def kernel(x, table):
    b, s = x.shape
    x_blocked = x.T.reshape(s * b // CHUNK, CHUNK).astype(jnp.int32)
    out5 = _make_kernel(b, s)(x_blocked, table)
    out = out5.transpose(2, 4, 0, 1, 3).reshape(b, s, D_MODEL)
    return out
